# baseline (speedup 1.0000x reference)
import functools

import jax
import jax.numpy as jnp
import numpy as np
from jax import lax
from jax.experimental import pallas as pl
from jax.experimental.pallas import tpu as pltpu
from jax.experimental.pallas import tpu_sc as plsc

B, S, D = 1024, 512, 64
N = B * S
FIELD_V, TIME_V = 50000, 10000

_info = plsc.get_sparse_core_info()
NC, NS, L = _info.num_cores, _info.num_subcores, _info.num_lanes
NW = NC * NS
PER_W = N // NW
K = 128
CHUNKS = PER_W // K
NBUF = 2

_POS_PERM = np.concatenate(
    [np.concatenate([g * 32 + np.arange(0, 32, 2), g * 32 + np.arange(1, 32, 2)])
     for g in range(D // 32)])

_mesh = plsc.VectorSubcoreMesh(core_axis_name="c", subcore_axis_name="s")

_scratch = (
    [pltpu.VMEM((K,), jnp.int32) for _ in range(5 * NBUF)]
    + [pltpu.VMEM((K, D), jnp.bfloat16) for _ in range(4 * NBUF)]
    + [
        pltpu.VMEM((D, K), jnp.float32),
        pltpu.VMEM((S, D), jnp.float32),
        pltpu.VMEM((4 * D,), jnp.float32),
        pltpu.VMEM_SHARED((TIME_V, D), jnp.bfloat16),
        pltpu.SemaphoreType.DMA,
        pltpu.SemaphoreType.DMA,
        pltpu.SemaphoreType.DMA,
        pltpu.SemaphoreType.DMA,
        pltpu.SemaphoreType.DMA,
        pltpu.SemaphoreType.DMA,
        pltpu.SemaphoreType.DMA,
    ]
)


@functools.partial(
    pl.kernel,
    mesh=_mesh,
    compiler_params=pltpu.CompilerParams(use_tc_tiling_on_sc=False,
                                         needs_layout_passes=False),
    out_type=jax.ShapeDtypeStruct((B, D // 8, S // K, 8, K), jnp.float32),
    scratch_types=_scratch,
)
def _emb_kernel(tok_i, typ_i, fld_i, ent_i, tim_i,
                tok_t, pos_t, typ_t, fld_t, ent_t, tim_t,
                out, *scr):
    iv = [scr[0:5], scr[5:10]]
    rv = [scr[10:14], scr[14:18]]
    out_v = scr[18]
    pos_v = scr[19]
    typ_v = scr[20]
    tim_sh = scr[21]
    sem_i = [scr[22], scr[23]]
    sem_g = [scr[24], scr[25]]
    sem_s = [scr[26], scr[27]]
    sem_st = scr[28]

    wid = lax.axis_index("s") * NC + lax.axis_index("c")
    base0 = wid * PER_W

    @pl.when(lax.axis_index("s") == 0)
    def _():
        pltpu.sync_copy(tim_t, tim_sh)

    pltpu.sync_copy(pos_t, pos_v)
    pltpu.sync_copy(typ_t, typ_v)
    plsc.subcore_barrier()

    def issue_idx(c, b):
        base = base0 + c * K
        for h, v in zip([tok_i, typ_i, fld_i, ent_i, tim_i], iv[b]):
            pltpu.async_copy(h.at[pl.ds(base, K)], v, sem_i[b])

    def wait_idx(b):
        for h, v in zip([tok_i, typ_i, fld_i, ent_i, tim_i], iv[b]):
            pltpu.make_async_copy(h.at[pl.ds(0, K)], v, sem_i[b]).wait()

    def issue_gathers(c, b):
        tok_iv, typ_iv, fld_iv, ent_iv, tim_iv = iv[b]
        tok_v, fld_v, ent_v, tim_v = rv[b]
        pltpu.async_copy(tok_t.at[tok_iv], tok_v, sem_g[b])
        pltpu.async_copy(ent_t.at[ent_iv], ent_v, sem_g[b])
        pltpu.async_copy(fld_t.at[fld_iv], fld_v, sem_g[b])
        pltpu.async_copy(tim_sh.at[tim_iv], tim_v, sem_s[b])

    def wait_gathers(b):
        tok_iv, typ_iv, fld_iv, ent_iv, tim_iv = iv[b]
        tok_v, fld_v, ent_v, tim_v = rv[b]
        pltpu.make_async_copy(tok_t.at[tok_iv], tok_v, sem_g[b]).wait()
        pltpu.make_async_copy(ent_t.at[ent_iv], ent_v, sem_g[b]).wait()
        pltpu.make_async_copy(fld_t.at[fld_iv], fld_v, sem_g[b]).wait()
        pltpu.make_async_copy(tim_sh.at[tim_iv], tim_v, sem_s[b]).wait()

    def issue_store(c):
        base = base0 + c * K
        bb = base // S
        jj = lax.rem(base, S) // K
        for i in range(D // 8):
            pltpu.async_copy(out_v.at[pl.ds(i * 8, 8), :],
                             out.at[bb, i, jj], sem_st)

    def wait_store():
        for i in range(D // 8):
            pltpu.make_async_copy(out_v.at[pl.ds(i * 8, 8), :],
                                  out.at[0, i, 0], sem_st).wait()

    iota = lax.iota(jnp.int32, L)
    d_even = [jnp.int32(g * 32) + 2 * iota for g in range(D // 32)]
    d_odd = [jnp.int32(g * 32 + 1) + 2 * iota for g in range(D // 32)]
    himask = jnp.full((L,), jnp.int32(-65536))

    def compute(c, b):
        pos_off = lax.rem(c, S // K) * K
        typ_iv = iv[b][1]
        tok_v, fld_v, ent_v, tim_v = rv[b]

        def row_body(p, carry):
            p_vec = jnp.full((L,), p, jnp.int32)
            t64 = lax.shift_left(plsc.load_gather(typ_iv, [p_vec]), 6)
            pp = pos_off + p
            for g in range(D // 32):
                sl = pl.ds(g * 32, 32)
                xs = [plsc.bitcast(v[p, sl], jnp.int32)
                      for v in (tok_v, fld_v, ent_v, tim_v)]
                typ_e = plsc.load_gather(typ_v, [t64 + d_even[g]])
                typ_o = plsc.load_gather(typ_v, [t64 + d_odd[g]])
                acc_e = pos_v[pp, pl.ds(g * 32, L)] + typ_e
                acc_o = pos_v[pp, pl.ds(g * 32 + L, L)] + typ_o
                for x in xs:
                    acc_e = acc_e + plsc.bitcast(lax.shift_left(x, 16), jnp.float32)
                    acc_o = acc_o + plsc.bitcast(x & himask, jnp.float32)
                plsc.store_scatter(out_v, [d_even[g], p_vec], acc_e)
                plsc.store_scatter(out_v, [d_odd[g], p_vec], acc_o)
            return carry

        lax.fori_loop(0, K, row_body, 0)

    issue_idx(0, 0)
    issue_idx(1, 1)
    wait_idx(0)
    issue_gathers(0, 0)

    def super_body(cc, carry):
        for b in range(NBUF):
            c = cc * NBUF + b
            nb = 1 - b

            @pl.when(c + 1 < CHUNKS)
            def _():
                wait_idx(nb)
                issue_gathers(c + 1, nb)

            wait_gathers(b)

            @pl.when(c >= 1)
            def _():
                wait_store()

            compute(c, b)
            issue_store(c)

            @pl.when(c + 2 < CHUNKS)
            def _():
                issue_idx(c + 2, b)
        return carry

    lax.fori_loop(0, CHUNKS // NBUF, super_body, 0)
    wait_store()


def kernel(token_ids, token_type_ids, field_ids, entity_ids, time_ids,
           token_table, pos_table, type_table, field_table, entity_table, time_table):
    tok = token_ids.reshape(-1).astype(jnp.int32)
    typ = token_type_ids.reshape(-1).astype(jnp.int32)
    fld = field_ids.reshape(-1).astype(jnp.int32)
    ent = entity_ids.reshape(-1).astype(jnp.int32)
    tim = time_ids.reshape(-1).astype(jnp.int32)
    out = _emb_kernel(tok, typ, fld, ent, tim,
                      token_table.astype(jnp.bfloat16),
                      pos_table[:, _POS_PERM],
                      type_table.reshape(-1),
                      field_table.astype(jnp.bfloat16),
                      entity_table.astype(jnp.bfloat16),
                      time_table.astype(jnp.bfloat16))
    return out.transpose(0, 2, 4, 1, 3).reshape(B, S, D)


# speedup vs baseline: 1.0039x; 1.0039x over previous
import functools

import jax
import jax.numpy as jnp
import numpy as np
from jax import lax
from jax.experimental import pallas as pl
from jax.experimental.pallas import tpu as pltpu
from jax.experimental.pallas import tpu_sc as plsc

B, S, D = 1024, 512, 64
N = B * S
FIELD_V, TIME_V = 50000, 10000

_info = plsc.get_sparse_core_info()
NC, NS, L = _info.num_cores, _info.num_subcores, _info.num_lanes
NW = NC * NS
PER_W = N // NW
K = 128
CHUNKS = PER_W // K
NBUF = 2

_POS_PERM = np.concatenate(
    [np.concatenate([g * 32 + np.arange(0, 32, 2), g * 32 + np.arange(1, 32, 2)])
     for g in range(D // 32)])

_mesh = plsc.VectorSubcoreMesh(core_axis_name="c", subcore_axis_name="s")

_scratch = (
    [pltpu.VMEM((K,), jnp.int32) for _ in range(5 * NBUF)]
    + [pltpu.VMEM((K, D), jnp.bfloat16) for _ in range(4 * NBUF)]
    + [
        pltpu.VMEM((D // 8, 8, K), jnp.float32),
        pltpu.VMEM((S, D), jnp.float32),
        pltpu.VMEM((4 * D,), jnp.float32),
        pltpu.VMEM_SHARED((TIME_V, D), jnp.bfloat16),
        pltpu.SemaphoreType.DMA,
        pltpu.SemaphoreType.DMA,
        pltpu.SemaphoreType.DMA,
        pltpu.SemaphoreType.DMA,
        pltpu.SemaphoreType.DMA,
        pltpu.SemaphoreType.DMA,
        pltpu.SemaphoreType.DMA,
    ]
)


@functools.partial(
    pl.kernel,
    mesh=_mesh,
    compiler_params=pltpu.CompilerParams(use_tc_tiling_on_sc=False,
                                         needs_layout_passes=False),
    out_type=jax.ShapeDtypeStruct((B, D // 8, S // K, 8, K), jnp.float32),
    scratch_types=_scratch,
)
def _emb_kernel(tok_i, typ_i, fld_i, ent_i, tim_i,
                tok_t, pos_t, typ_t, fld_t, ent_t, tim_t,
                out, *scr):
    iv = [scr[0:5], scr[5:10]]
    rv = [scr[10:14], scr[14:18]]
    out_v = scr[18]
    pos_v = scr[19]
    typ_v = scr[20]
    tim_sh = scr[21]
    sem_i = [scr[22], scr[23]]
    sem_g = [scr[24], scr[25]]
    sem_s = [scr[26], scr[27]]
    sem_st = scr[28]

    wid = lax.axis_index("s") * NC + lax.axis_index("c")
    base0 = wid * PER_W

    @pl.when(lax.axis_index("s") == 0)
    def _():
        pltpu.sync_copy(tim_t, tim_sh)

    pltpu.sync_copy(pos_t, pos_v)
    pltpu.sync_copy(typ_t, typ_v)
    plsc.subcore_barrier()

    def issue_idx(c, b):
        base = base0 + c * K
        for h, v in zip([tok_i, typ_i, fld_i, ent_i, tim_i], iv[b]):
            pltpu.async_copy(h.at[pl.ds(base, K)], v, sem_i[b])

    def wait_idx(b):
        for h, v in zip([tok_i, typ_i, fld_i, ent_i, tim_i], iv[b]):
            pltpu.make_async_copy(h.at[pl.ds(0, K)], v, sem_i[b]).wait()

    def issue_gathers(c, b):
        tok_iv, typ_iv, fld_iv, ent_iv, tim_iv = iv[b]
        tok_v, fld_v, ent_v, tim_v = rv[b]
        pltpu.async_copy(tok_t.at[tok_iv], tok_v, sem_g[b])
        pltpu.async_copy(ent_t.at[ent_iv], ent_v, sem_g[b])
        pltpu.async_copy(fld_t.at[fld_iv], fld_v, sem_g[b])
        pltpu.async_copy(tim_sh.at[tim_iv], tim_v, sem_s[b])

    def wait_gathers(b):
        tok_iv, typ_iv, fld_iv, ent_iv, tim_iv = iv[b]
        tok_v, fld_v, ent_v, tim_v = rv[b]
        pltpu.make_async_copy(tok_t.at[tok_iv], tok_v, sem_g[b]).wait()
        pltpu.make_async_copy(ent_t.at[ent_iv], ent_v, sem_g[b]).wait()
        pltpu.make_async_copy(fld_t.at[fld_iv], fld_v, sem_g[b]).wait()
        pltpu.make_async_copy(tim_sh.at[tim_iv], tim_v, sem_s[b]).wait()

    def issue_store(c):
        base = base0 + c * K
        bb = base // S
        jj = lax.rem(base, S) // K
        pltpu.async_copy(out_v, out.at[bb, :, jj], sem_st)

    def wait_store():
        pltpu.make_async_copy(out_v, out.at[0, :, 0], sem_st).wait()

    iota = lax.iota(jnp.int32, L)
    d_even = [jnp.int32(g * 32) + 2 * iota for g in range(D // 32)]
    d_odd = [jnp.int32(g * 32 + 1) + 2 * iota for g in range(D // 32)]
    d_even_hi = [lax.shift_right_logical(d, 3) for d in d_even]
    d_even_lo = [d & jnp.full((L,), jnp.int32(7)) for d in d_even]
    d_odd_hi = [lax.shift_right_logical(d, 3) for d in d_odd]
    d_odd_lo = [d & jnp.full((L,), jnp.int32(7)) for d in d_odd]
    himask = jnp.full((L,), jnp.int32(-65536))

    def compute(c, b):
        pos_off = lax.rem(c, S // K) * K
        typ_iv = iv[b][1]
        tok_v, fld_v, ent_v, tim_v = rv[b]

        def row_body(p, carry):
            p_vec = jnp.full((L,), p, jnp.int32)
            t64 = lax.shift_left(plsc.load_gather(typ_iv, [p_vec]), 6)
            pp = pos_off + p
            for g in range(D // 32):
                sl = pl.ds(g * 32, 32)
                xs = [plsc.bitcast(v[p, sl], jnp.int32)
                      for v in (tok_v, fld_v, ent_v, tim_v)]
                typ_e = plsc.load_gather(typ_v, [t64 + d_even[g]])
                typ_o = plsc.load_gather(typ_v, [t64 + d_odd[g]])
                acc_e = pos_v[pp, pl.ds(g * 32, L)] + typ_e
                acc_o = pos_v[pp, pl.ds(g * 32 + L, L)] + typ_o
                for x in xs:
                    acc_e = acc_e + plsc.bitcast(lax.shift_left(x, 16), jnp.float32)
                    acc_o = acc_o + plsc.bitcast(x & himask, jnp.float32)
                plsc.store_scatter(out_v, [d_even_hi[g], d_even_lo[g], p_vec], acc_e)
                plsc.store_scatter(out_v, [d_odd_hi[g], d_odd_lo[g], p_vec], acc_o)
            return carry

        lax.fori_loop(0, K, row_body, 0)

    issue_idx(0, 0)
    issue_idx(1, 1)
    wait_idx(0)
    issue_gathers(0, 0)

    def super_body(cc, carry):
        for b in range(NBUF):
            c = cc * NBUF + b
            nb = 1 - b

            @pl.when(c + 1 < CHUNKS)
            def _():
                wait_idx(nb)
                issue_gathers(c + 1, nb)

            wait_gathers(b)

            @pl.when(c >= 1)
            def _():
                wait_store()

            compute(c, b)
            issue_store(c)

            @pl.when(c + 2 < CHUNKS)
            def _():
                issue_idx(c + 2, b)
        return carry

    lax.fori_loop(0, CHUNKS // NBUF, super_body, 0)
    wait_store()


def kernel(token_ids, token_type_ids, field_ids, entity_ids, time_ids,
           token_table, pos_table, type_table, field_table, entity_table, time_table):
    tok = token_ids.reshape(-1).astype(jnp.int32)
    typ = token_type_ids.reshape(-1).astype(jnp.int32)
    fld = field_ids.reshape(-1).astype(jnp.int32)
    ent = entity_ids.reshape(-1).astype(jnp.int32)
    tim = time_ids.reshape(-1).astype(jnp.int32)
    out = _emb_kernel(tok, typ, fld, ent, tim,
                      token_table.astype(jnp.bfloat16),
                      pos_table[:, _POS_PERM],
                      type_table.reshape(-1),
                      field_table.astype(jnp.bfloat16),
                      entity_table.astype(jnp.bfloat16),
                      time_table.astype(jnp.bfloat16))
    return out.transpose(0, 2, 4, 1, 3).reshape(B, S, D)


# speedup vs baseline: 1.2193x; 1.2146x over previous
import functools

import jax
import jax.numpy as jnp
import numpy as np
from jax import lax
from jax.experimental import pallas as pl
from jax.experimental.pallas import tpu as pltpu
from jax.experimental.pallas import tpu_sc as plsc

B, S, D = 1024, 512, 64
N = B * S
TYPE_V, TIME_V = 4, 10000

_info = plsc.get_sparse_core_info()
NC, NS, L = _info.num_cores, _info.num_subcores, _info.num_lanes
NW = NC * NS
PER_W = N // NW
K = 128
CHUNKS = PER_W // K
NBUF = 2
NT = 5

_POS_PERM = np.concatenate(
    [np.concatenate([g * 32 + np.arange(0, 32, 2), g * 32 + np.arange(1, 32, 2)])
     for g in range(D // 32)])

_mesh = plsc.VectorSubcoreMesh(core_axis_name="c", subcore_axis_name="s")

_scratch = (
    [pltpu.VMEM((K,), jnp.int32) for _ in range(NT * NBUF)]
    + [pltpu.VMEM((K, D), jnp.bfloat16) for _ in range(NT * NBUF)]
    + [
        pltpu.VMEM((K, D), jnp.float32),
        pltpu.VMEM((S, D), jnp.float32),
        pltpu.VMEM_SHARED((TYPE_V, D), jnp.bfloat16),
        pltpu.VMEM_SHARED((TIME_V, D), jnp.bfloat16),
        pltpu.SemaphoreType.DMA,
        pltpu.SemaphoreType.DMA,
        pltpu.SemaphoreType.DMA,
        pltpu.SemaphoreType.DMA,
        pltpu.SemaphoreType.DMA,
        pltpu.SemaphoreType.DMA,
        pltpu.SemaphoreType.DMA,
    ]
)


@functools.partial(
    pl.kernel,
    mesh=_mesh,
    compiler_params=pltpu.CompilerParams(use_tc_tiling_on_sc=False,
                                         needs_layout_passes=False),
    out_type=jax.ShapeDtypeStruct((N, D), jnp.float32),
    scratch_types=_scratch,
)
def _emb_kernel(tok_i, typ_i, fld_i, ent_i, tim_i,
                tok_t, pos_t, typ_t, fld_t, ent_t, tim_t,
                out, *scr):
    iv = [scr[0:NT], scr[NT:2 * NT]]
    rv = [scr[10:10 + NT], scr[10 + NT:10 + 2 * NT]]
    out_v = scr[20]
    pos_v = scr[21]
    typ_sh = scr[22]
    tim_sh = scr[23]
    sem_i = [scr[24], scr[25]]
    sem_g = [scr[26], scr[27]]
    sem_s = [scr[28], scr[29]]
    sem_st = scr[30]

    wid = lax.axis_index("s") * NC + lax.axis_index("c")
    base0 = wid * PER_W

    @pl.when(lax.axis_index("s") == 0)
    def _():
        pltpu.sync_copy(typ_t, typ_sh)
        pltpu.sync_copy(tim_t, tim_sh)

    pltpu.sync_copy(pos_t, pos_v)
    plsc.subcore_barrier()

    def issue_idx(c, b):
        base = base0 + c * K
        for h, v in zip([tok_i, typ_i, fld_i, ent_i, tim_i], iv[b]):
            pltpu.async_copy(h.at[pl.ds(base, K)], v, sem_i[b])

    def wait_idx(b):
        for h, v in zip([tok_i, typ_i, fld_i, ent_i, tim_i], iv[b]):
            pltpu.make_async_copy(h.at[pl.ds(0, K)], v, sem_i[b]).wait()

    def issue_gathers(c, b):
        tok_iv, typ_iv, fld_iv, ent_iv, tim_iv = iv[b]
        tok_v, typ_v, fld_v, ent_v, tim_v = rv[b]
        pltpu.async_copy(tok_t.at[tok_iv], tok_v, sem_g[b])
        pltpu.async_copy(ent_t.at[ent_iv], ent_v, sem_g[b])
        pltpu.async_copy(fld_t.at[fld_iv], fld_v, sem_g[b])
        pltpu.async_copy(typ_sh.at[typ_iv], typ_v, sem_s[b])
        pltpu.async_copy(tim_sh.at[tim_iv], tim_v, sem_s[b])

    def wait_gathers(b):
        tok_iv, typ_iv, fld_iv, ent_iv, tim_iv = iv[b]
        tok_v, typ_v, fld_v, ent_v, tim_v = rv[b]
        pltpu.make_async_copy(tok_t.at[tok_iv], tok_v, sem_g[b]).wait()
        pltpu.make_async_copy(ent_t.at[ent_iv], ent_v, sem_g[b]).wait()
        pltpu.make_async_copy(fld_t.at[fld_iv], fld_v, sem_g[b]).wait()
        pltpu.make_async_copy(typ_sh.at[typ_iv], typ_v, sem_s[b]).wait()
        pltpu.make_async_copy(tim_sh.at[tim_iv], tim_v, sem_s[b]).wait()

    def wait_store():
        pltpu.make_async_copy(out_v, out.at[pl.ds(base0, K)], sem_st).wait()

    iota = lax.iota(jnp.int32, L)
    d_even = [jnp.int32(g * 32) + 2 * iota for g in range(D // 32)]
    d_odd = [jnp.int32(g * 32 + 1) + 2 * iota for g in range(D // 32)]
    himask = jnp.full((L,), jnp.int32(-65536))

    def compute(c, b):
        pos_off = lax.rem(c, S // K) * K
        bufs = rv[b]

        def row_body(p, carry):
            p_vec = jnp.full((L,), p, jnp.int32)
            pp = pos_off + p
            for g in range(D // 32):
                sl = pl.ds(g * 32, 32)
                xs = [plsc.bitcast(v[p, sl], jnp.int32) for v in bufs]
                acc_e = pos_v[pp, pl.ds(g * 32, L)]
                acc_o = pos_v[pp, pl.ds(g * 32 + L, L)]
                for x in xs:
                    acc_e = acc_e + plsc.bitcast(lax.shift_left(x, 16), jnp.float32)
                    acc_o = acc_o + plsc.bitcast(x & himask, jnp.float32)
                plsc.store_scatter(out_v, [p_vec, d_even[g]], acc_e)
                plsc.store_scatter(out_v, [p_vec, d_odd[g]], acc_o)
            return carry

        lax.fori_loop(0, K, row_body, 0)

    issue_idx(0, 0)
    issue_idx(1, 1)
    wait_idx(0)
    issue_gathers(0, 0)

    def super_body(cc, carry):
        for b in range(NBUF):
            c = cc * NBUF + b
            nb = 1 - b

            @pl.when(c + 1 < CHUNKS)
            def _():
                wait_idx(nb)
                issue_gathers(c + 1, nb)

            wait_gathers(b)

            @pl.when(c + 2 < CHUNKS)
            def _():
                issue_idx(c + 2, b)

            @pl.when(c >= 1)
            def _():
                wait_store()

            compute(c, b)
            pltpu.async_copy(out_v, out.at[pl.ds(base0 + c * K, K)], sem_st)
        return carry

    lax.fori_loop(0, CHUNKS // NBUF, super_body, 0)
    wait_store()


def kernel(token_ids, token_type_ids, field_ids, entity_ids, time_ids,
           token_table, pos_table, type_table, field_table, entity_table, time_table):
    tok = token_ids.reshape(-1).astype(jnp.int32)
    typ = token_type_ids.reshape(-1).astype(jnp.int32)
    fld = field_ids.reshape(-1).astype(jnp.int32)
    ent = entity_ids.reshape(-1).astype(jnp.int32)
    tim = time_ids.reshape(-1).astype(jnp.int32)
    out = _emb_kernel(tok, typ, fld, ent, tim,
                      token_table.astype(jnp.bfloat16),
                      pos_table[:, _POS_PERM],
                      type_table.astype(jnp.bfloat16),
                      field_table.astype(jnp.bfloat16),
                      entity_table.astype(jnp.bfloat16),
                      time_table.astype(jnp.bfloat16))
    return out.reshape(B, S, D)
